# Initial kernel scaffold; baseline (speedup 1.0000x reference)
import jax
import jax.numpy as jnp
from jax.experimental import pallas as pl


def kernel(diag_param, tril_param):
    raise NotImplementedError("write your pallas kernel here")


# speedup vs baseline: 173.4842x; 173.4842x over previous
import functools

import jax
import jax.numpy as jnp
from jax import lax
from jax.experimental import pallas as pl
from jax.experimental.pallas import tpu as pltpu
from jax.experimental.pallas import tpu_sc as plsc

SIZE = 4096
TRIL_SIZE = SIZE * (SIZE - 1) // 2
NC = 2
NS = 16
NW = NC * NS
ROWS_PER_W = SIZE // NW
C = 512
L = 16
NVREG = SIZE // L
TMP_WORDS = SIZE + C


def _body(diag_hbm, tril_hbm, theta_hbm, csum_hbm, tmp, outbuf, diag_v, cvec):
    cid = lax.axis_index("c")
    sid = lax.axis_index("s")
    wid = sid * NC + cid

    pltpu.sync_copy(diag_hbm, diag_v)

    iota = lax.iota(jnp.int32, L)

    def do_row(r, _):
        i = r * NW + wid
        t = (i * (i - 1)) // 2
        a = (t // 8) * 8
        s = t - a
        nread = (s + i + C - 1) // C

        def rd(k, _):
            pltpu.sync_copy(tril_hbm.at[pl.ds(a + k * C, C)],
                            tmp.at[pl.ds(k * C, C)])
            return _

        lax.fori_loop(0, nread, rd, None)

        dchunk = jnp.exp(diag_v[pl.ds((i // L) * L, L)])

        def vx(j, _):
            col0 = j * L
            cols = col0 + iota
            v = tmp[pl.ds(s + col0, L)]
            r1 = jnp.where(cols < i, v, jnp.float32(0.0))
            r2 = jnp.where(cols == i, dchunk, r1)
            outbuf[pl.ds(col0, L)] = r2
            return _

        lax.fori_loop(0, NVREG, vx, None)

        pltpu.sync_copy(outbuf, theta_hbm.at[i])
        return _

    lax.fori_loop(0, ROWS_PER_W, do_row, None)

    @pl.when(wid == 0)
    def _():
        def acc_fn(k, acc):
            return acc + diag_v[pl.ds(k * L, L)]

        acc = lax.fori_loop(0, NVREG, acc_fn, jnp.zeros((L,), jnp.float32))
        dnums = lax.GatherDimensionNumbers(
            offset_dims=(), collapsed_slice_dims=(0,), start_index_map=(0,))
        for d in (8, 4, 2, 1):
            perm = lax.gather(acc, (iota ^ d)[:, None], dnums, (1,),
                              mode=lax.GatherScatterMode.PROMISE_IN_BOUNDS)
            acc = acc + perm
        cvec[...] = acc
        pltpu.sync_copy(cvec, csum_hbm)


@jax.jit
def _build(diag_param, tril_param):
    mesh = plsc.VectorSubcoreMesh(core_axis_name="c", subcore_axis_name="s",
                                  num_cores=NC, num_subcores=NS)
    theta, csum = pl.kernel(
        _body,
        out_type=(
            jax.ShapeDtypeStruct((SIZE, SIZE), jnp.float32),
            jax.ShapeDtypeStruct((L,), jnp.float32),
        ),
        mesh=mesh,
        scratch_types=(
            pltpu.VMEM((TMP_WORDS,), jnp.float32),
            pltpu.VMEM((SIZE,), jnp.float32),
            pltpu.VMEM((SIZE,), jnp.float32),
            pltpu.VMEM((L,), jnp.float32),
        ),
    )(diag_param, tril_param)
    return theta, csum[0]


def kernel(diag_param, tril_param):
    return _build(diag_param, tril_param)


# speedup vs baseline: 747.9049x; 4.3111x over previous
import jax
import jax.numpy as jnp
from jax import lax
from jax.experimental import pallas as pl
from jax.experimental.pallas import tpu as pltpu
from jax.experimental.pallas import tpu_sc as plsc

SIZE = 4096
TRIL_SIZE = SIZE * (SIZE - 1) // 2
NC = 2
NS = 16
NW = NC * NS
ROWS_PER_W = SIZE // NW
C = 512
L = 16
BLK = 128
NBUF = 4
TMP_WORDS = SIZE + C + 64


def _tri(i):
    return (i * (i - 1)) // 2


def _body(diag_hbm, tril_hbm, theta_hbm, csum_hbm,
          diag_v, cvec, bufs, tmps, rd_sems, wr_sems):
    cid = lax.axis_index("c")
    sid = lax.axis_index("s")
    wid = sid * NC + cid

    pltpu.sync_copy(diag_hbm, diag_v)
    iota = lax.iota(jnp.int32, L)

    def row_of(k):
        return k * NW + wid

    def nread_of(i):
        t = _tri(i)
        s = t - (t // 8) * 8
        return (s + i + C - 1) // C

    def issue_reads(i, tmp, sem):
        t = _tri(i)
        a = (t // 8) * 8

        def rd(k, _):
            pltpu.async_copy(tril_hbm.at[pl.ds(a + k * C, C)],
                             tmp.at[pl.ds(k * C, C)], sem)
            return _

        lax.fori_loop(0, nread_of(i), rd, None)

    def wait_reads(i, tmp, sem):
        def wt(k, _):
            pltpu.make_async_copy(tril_hbm.at[pl.ds(0, C)],
                                  tmp.at[pl.ds(0, C)], sem).wait()
            return _

        lax.fori_loop(0, nread_of(i), wt, None)

    def build_row(i, buf, tmp):
        t = _tri(i)
        s = t - (t // 8) * 8
        g0 = i // BLK

        def blk(g, _):
            base = g * BLK
            for jj in range(BLK // L):
                buf[pl.ds(base + jj * L, L)] = tmp[pl.ds(s + base + jj * L, L)]
            return _

        lax.fori_loop(0, g0, blk, None)

        dchunk = jnp.exp(diag_v[pl.ds((i // L) * L, L)])
        base = g0 * BLK
        for jj in range(BLK // L):
            cols = base + jj * L + iota
            v = tmp[pl.ds(s + base + jj * L, L)]
            r = jnp.where(cols < i, v, jnp.float32(0.0))
            r = jnp.where(cols == i, dchunk, r)
            buf[pl.ds(base + jj * L, L)] = r

    def wait_write(sem):
        pltpu.make_async_copy(theta_hbm.at[0], bufs[0], sem).wait()

    zeros16 = jnp.zeros((L,), jnp.float32)

    def z(j, _):
        for b in range(NBUF):
            bufs[b][pl.ds(j * L, L)] = zeros16
        return _

    lax.fori_loop(0, SIZE // L, z, None)

    issue_reads(row_of(0), tmps[0], rd_sems[0])

    def step(k, _):
        for b in range(NBUF):
            @pl.when(k % NBUF == b)
            def _():
                nb = (b + 1) % NBUF

                @pl.when(k >= NBUF - 1)
                def _():
                    wait_write(wr_sems[nb])

                @pl.when(k + 1 < ROWS_PER_W)
                def _():
                    issue_reads(row_of(k + 1), tmps[nb], rd_sems[nb])

                i = row_of(k)
                wait_reads(i, tmps[b], rd_sems[b])
                build_row(i, bufs[b], tmps[b])
                pltpu.async_copy(bufs[b], theta_hbm.at[i], wr_sems[b])
        return _

    lax.fori_loop(0, ROWS_PER_W, step, None)
    for r in range(ROWS_PER_W - NBUF + 1, ROWS_PER_W):
        wait_write(wr_sems[r % NBUF])

    @pl.when(wid == 0)
    def _():
        def acc_fn(k, acc):
            return acc + diag_v[pl.ds(k * L, L)]

        acc = lax.fori_loop(0, SIZE // L, acc_fn,
                            jnp.zeros((L,), jnp.float32))
        dnums = lax.GatherDimensionNumbers(
            offset_dims=(), collapsed_slice_dims=(0,), start_index_map=(0,))
        for d in (8, 4, 2, 1):
            perm = lax.gather(acc, (iota ^ d)[:, None], dnums, (1,),
                              mode=lax.GatherScatterMode.PROMISE_IN_BOUNDS)
            acc = acc + perm
        cvec[...] = acc
        pltpu.sync_copy(cvec, csum_hbm)


@jax.jit
def _build(diag_param, tril_param):
    mesh = plsc.VectorSubcoreMesh(core_axis_name="c", subcore_axis_name="s",
                                  num_cores=NC, num_subcores=NS)
    theta, csum = pl.kernel(
        _body,
        out_type=(
            jax.ShapeDtypeStruct((SIZE, SIZE), jnp.float32),
            jax.ShapeDtypeStruct((L,), jnp.float32),
        ),
        mesh=mesh,
        scratch_types=(
            pltpu.VMEM((SIZE,), jnp.float32),
            pltpu.VMEM((L,), jnp.float32),
            tuple(pltpu.VMEM((SIZE,), jnp.float32) for _ in range(NBUF)),
            tuple(pltpu.VMEM((TMP_WORDS,), jnp.float32) for _ in range(NBUF)),
            tuple(pltpu.SemaphoreType.DMA for _ in range(NBUF)),
            tuple(pltpu.SemaphoreType.DMA for _ in range(NBUF)),
        ),
    )(diag_param, tril_param)
    return theta, csum[0]


def kernel(diag_param, tril_param):
    return _build(diag_param, tril_param)
